# baseline (speedup 1.0000x reference)
import functools

import jax
import jax.numpy as jnp
from jax import lax
from jax.experimental import pallas as pl
from jax.experimental.pallas import tpu as pltpu
from jax.experimental.pallas import tpu_sc as plsc

_N, _D, _H, _O, _B = 100000, 128, 128, 128, 1024
_BN = 2000

_NSLICES = 2
_SLICE = _N // _NSLICES
_CH = 128
_NW = 32
_STRIPE = _B // 16

_NFULL = _SLICE // _CH
_REM = _SLICE - _NFULL * _CH
_TMAX = -(-_NFULL // _NW)
_NTAIL = _NFULL % _NW


def _mlp_body(x_ref, w1_ref, b1_ref, w2_ref, b2_ref, h_ref):
    x = x_ref[...]
    h = jnp.dot(x, w1_ref[...], preferred_element_type=jnp.float32)
    h = jnp.maximum(h + b1_ref[...], 0.0)
    h = jnp.dot(h, w2_ref[...], preferred_element_type=jnp.float32)
    h_ref[...] = jnp.maximum(h + b2_ref[...], 0.0)


def _mlp(x, W1, b1, W2, b2):
    nb = x.shape[0] // _BN
    return pl.pallas_call(
        _mlp_body,
        grid=(nb,),
        in_specs=[
            pl.BlockSpec((_BN, _D), lambda g: (g, 0)),
            pl.BlockSpec((_D, _H), lambda g: (0, 0)),
            pl.BlockSpec((1, _H), lambda g: (0, 0)),
            pl.BlockSpec((_H, _H), lambda g: (0, 0)),
            pl.BlockSpec((1, _H), lambda g: (0, 0)),
        ],
        out_specs=pl.BlockSpec((_BN, _H), lambda g: (g, 0)),
        out_shape=jax.ShapeDtypeStruct((x.shape[0], _H), jnp.float32),
    )(x, W1, b1.reshape(1, _H), W2, b2.reshape(1, _H))


def _segsum_body(base_row, h_hbm, ids_hbm, zeros_hbm, out_hbm,
                 idx0, idx1, rows0, rows1, idx_r, rows_r, acc_sh,
                 sem0, sem1):
    cid = lax.axis_index("c")
    sid = lax.axis_index("s")
    wid = sid * 2 + cid
    idx = (idx0, idx1)
    rows = (rows0, rows1)
    sem = (sem0, sem1)

    pltpu.sync_copy(zeros_hbm.at[pl.ds(sid * _STRIPE, _STRIPE)],
                    acc_sh.at[pl.ds(sid * _STRIPE, _STRIPE)])

    def start(t, b):
        base = (wid + _NW * t) * _CH
        pltpu.async_copy(ids_hbm.at[pl.ds(base_row + base, _CH)],
                         idx[b], sem[b])
        pltpu.async_copy(h_hbm.at[pl.ds(base, _CH)], rows[b], sem[b])

    def finish(t, b):
        base = (wid + _NW * t) * _CH
        pltpu.make_async_copy(ids_hbm.at[pl.ds(base_row + base, _CH)],
                              idx[b], sem[b]).wait()
        pltpu.make_async_copy(h_hbm.at[pl.ds(base, _CH)], rows[b],
                              sem[b]).wait()
        pltpu.sync_copy(rows[b], acc_sh.at[idx[b]], add=True)

    start(0, 0)
    plsc.subcore_barrier()

    for t in range(_TMAX):
        b = t & 1
        if t + 1 < _TMAX - 1:
            start(t + 1, 1 - b)
        elif t + 1 == _TMAX - 1:
            @pl.when(wid < _NTAIL)
            def _start_tail():
                start(_TMAX - 1, 1 - b)
        if t < _TMAX - 1:
            finish(t, b)
        else:
            @pl.when(wid < _NTAIL)
            def _finish_tail():
                finish(_TMAX - 1, b)

    @pl.when(wid == _NW - 1)
    def _rem():
        base = _NFULL * _CH
        pltpu.sync_copy(ids_hbm.at[pl.ds(base_row + base, _REM)], idx_r)
        pltpu.sync_copy(h_hbm.at[pl.ds(base, _REM)], rows_r)
        pltpu.sync_copy(rows_r, acc_sh.at[idx_r], add=True)

    plsc.subcore_barrier()
    out_base = cid * _B + sid * _STRIPE
    pltpu.sync_copy(acc_sh.at[pl.ds(sid * _STRIPE, _STRIPE)],
                    out_hbm.at[pl.ds(out_base, _STRIPE)])


def _segsum(h, ids, zeros, base_row):
    mesh = plsc.VectorSubcoreMesh(core_axis_name="c", subcore_axis_name="s")
    f = functools.partial(
        pl.kernel,
        mesh=mesh,
        out_type=jax.ShapeDtypeStruct((2 * _B, _H), jnp.float32),
        scratch_types=[
            pltpu.VMEM((_CH,), jnp.int32),
            pltpu.VMEM((_CH,), jnp.int32),
            pltpu.VMEM((_CH, _H), jnp.float32),
            pltpu.VMEM((_CH, _H), jnp.float32),
            pltpu.VMEM((_REM,), jnp.int32),
            pltpu.VMEM((_REM, _H), jnp.float32),
            pltpu.VMEM_SHARED((_B, _H), jnp.float32),
            pltpu.SemaphoreType.DMA,
            pltpu.SemaphoreType.DMA,
        ],
    )(functools.partial(_segsum_body, base_row))
    return f(h, ids, zeros)


def _out_body(p0_ref, p1_ref, wout_ref, bout_ref, out_ref):
    acc = (p0_ref[0:_B, :] + p0_ref[_B:2 * _B, :]
           + p1_ref[0:_B, :] + p1_ref[_B:2 * _B, :])
    out_ref[...] = (jnp.dot(acc, wout_ref[...],
                            preferred_element_type=jnp.float32)
                    + bout_ref[...])


def _out_layer(partials, Wout, bout):
    return pl.pallas_call(
        _out_body,
        in_specs=[
            pl.BlockSpec((2 * _B, _H), lambda: (0, 0)),
            pl.BlockSpec((2 * _B, _H), lambda: (0, 0)),
            pl.BlockSpec((_H, _O), lambda: (0, 0)),
            pl.BlockSpec((1, _O), lambda: (0, 0)),
        ],
        out_specs=pl.BlockSpec((_B, _O), lambda: (0, 0)),
        out_shape=jax.ShapeDtypeStruct((_B, _O), jnp.float32),
    )(partials[0], partials[1], Wout, bout.reshape(1, _O))


def kernel(node_features, batch_vector, W1, b1, W2, b2, Wout, bout):
    ids = batch_vector.astype(jnp.int32)
    zeros = jnp.zeros((_B, _H), jnp.float32)
    partials = []
    for s in range(_NSLICES):
        x_s = lax.slice_in_dim(node_features, s * _SLICE, (s + 1) * _SLICE)
        h_s = _mlp(x_s, W1, b1, W2, b2)
        partials.append(_segsum(h_s, ids, zeros, s * _SLICE))
    return _out_layer(partials, Wout, bout)


# speedup vs baseline: 1.0084x; 1.0084x over previous
import functools

import jax
import jax.numpy as jnp
from jax import lax
from jax.experimental import pallas as pl
from jax.experimental.pallas import tpu as pltpu
from jax.experimental.pallas import tpu_sc as plsc

_N, _D, _H, _O, _B = 100000, 128, 128, 128, 1024
_BN = 1000
_SLICES = (25000, 25000, 25000, 25000)
_CH = 128
_NW = 32
_STRIPE = _B // 16
_NBUF = 3


def _mlp_body(x_ref, w1_ref, b1_ref, w2_ref, b2_ref, h_ref):
    x = x_ref[...]
    h = jnp.dot(x, w1_ref[...], preferred_element_type=jnp.float32)
    h = jnp.maximum(h + b1_ref[...], 0.0)
    h = jnp.dot(h, w2_ref[...], preferred_element_type=jnp.float32)
    h_ref[...] = jnp.maximum(h + b2_ref[...], 0.0)


def _mlp(x, W1, b1, W2, b2, base_blk, nb):
    return pl.pallas_call(
        _mlp_body,
        grid=(nb,),
        in_specs=[
            pl.BlockSpec((_BN, _D), lambda g: (g + base_blk, 0)),
            pl.BlockSpec((_D, _H), lambda g: (0, 0)),
            pl.BlockSpec((1, _H), lambda g: (0, 0)),
            pl.BlockSpec((_H, _H), lambda g: (0, 0)),
            pl.BlockSpec((1, _H), lambda g: (0, 0)),
        ],
        out_specs=pl.BlockSpec((_BN, _H), lambda g: (g, 0)),
        out_shape=jax.ShapeDtypeStruct((nb * _BN, _H), jnp.float32),
    )(x, W1, b1.reshape(1, _H), W2, b2.reshape(1, _H))


def _make_segsum_body(base_row, nrows, rem):
    nfull = nrows // _CH
    tmax = -(-nfull // _NW)
    ntail = nfull % _NW

    def body(h_hbm, ids_hbm, zeros_hbm, out_hbm,
             idx0, idx1, idx2, rows0, rows1, rows2, idx_r, rows_r, acc_sh,
             sl0, sl1, sl2, ss0, ss1, ss2):
        cid = lax.axis_index("c")
        sid = lax.axis_index("s")
        wid = sid * 2 + cid
        idx = (idx0, idx1, idx2)
        rows = (rows0, rows1, rows2)
        sem_l = (sl0, sl1, sl2)
        sem_s = (ss0, ss1, ss2)

        pltpu.sync_copy(zeros_hbm.at[pl.ds(sid * _STRIPE, _STRIPE)],
                        acc_sh.at[pl.ds(sid * _STRIPE, _STRIPE)])

        def valid(t):
            return t < tmax - 1 or ntail == 0

        def guard(t, fn):
            if valid(t):
                fn()
            else:
                pl.when(wid < ntail)(fn)

        def start_load(t, b):
            base = (wid + _NW * t) * _CH
            pltpu.async_copy(ids_hbm.at[pl.ds(base_row + base, _CH)],
                             idx[b], sem_l[b])
            pltpu.async_copy(h_hbm.at[pl.ds(base, _CH)], rows[b], sem_l[b])

        def issue_scatter(t, b):
            base = (wid + _NW * t) * _CH
            pltpu.make_async_copy(ids_hbm.at[pl.ds(base_row + base, _CH)],
                                  idx[b], sem_l[b]).wait()
            pltpu.make_async_copy(h_hbm.at[pl.ds(base, _CH)], rows[b],
                                  sem_l[b]).wait()
            pltpu.async_copy(rows[b], acc_sh.at[idx[b]], sem_s[b], add=True)

        def wait_scatter(t, b):
            pltpu.make_async_copy(rows[b], acc_sh.at[idx[b]],
                                  sem_s[b]).wait()

        guard(0, lambda: start_load(0, 0))
        plsc.subcore_barrier()

        for t in range(tmax):
            b = t % _NBUF
            if t >= 2:
                guard(t - 2, functools.partial(wait_scatter, t - 2,
                                               (t - 2) % _NBUF))
            if t + 1 < tmax:
                guard(t + 1, functools.partial(start_load, t + 1,
                                               (t + 1) % _NBUF))
            guard(t, functools.partial(issue_scatter, t, b))
        for t in (tmax - 2, tmax - 1):
            if t >= 0:
                guard(t, functools.partial(wait_scatter, t, t % _NBUF))

        if rem > 0:
            @pl.when(wid == _NW - 1)
            def _rem():
                base = nfull * _CH
                pltpu.sync_copy(ids_hbm.at[pl.ds(base_row + base, rem)],
                                idx_r)
                pltpu.sync_copy(h_hbm.at[pl.ds(base, rem)], rows_r)
                pltpu.sync_copy(rows_r, acc_sh.at[idx_r], add=True)

        plsc.subcore_barrier()
        out_base = cid * _B + sid * _STRIPE
        pltpu.sync_copy(acc_sh.at[pl.ds(sid * _STRIPE, _STRIPE)],
                        out_hbm.at[pl.ds(out_base, _STRIPE)])

    return body


def _segsum(h, ids, zeros, base_row, nrows):
    rem = nrows % _CH
    mesh = plsc.VectorSubcoreMesh(core_axis_name="c", subcore_axis_name="s")
    f = functools.partial(
        pl.kernel,
        mesh=mesh,
        out_type=jax.ShapeDtypeStruct((2 * _B, _H), jnp.float32),
        scratch_types=[
            pltpu.VMEM((_CH,), jnp.int32),
            pltpu.VMEM((_CH,), jnp.int32),
            pltpu.VMEM((_CH,), jnp.int32),
            pltpu.VMEM((_CH, _H), jnp.float32),
            pltpu.VMEM((_CH, _H), jnp.float32),
            pltpu.VMEM((_CH, _H), jnp.float32),
            pltpu.VMEM((max(rem, 8),), jnp.int32),
            pltpu.VMEM((max(rem, 8), _H), jnp.float32),
            pltpu.VMEM_SHARED((_B, _H), jnp.float32),
            pltpu.SemaphoreType.DMA,
            pltpu.SemaphoreType.DMA,
            pltpu.SemaphoreType.DMA,
            pltpu.SemaphoreType.DMA,
            pltpu.SemaphoreType.DMA,
            pltpu.SemaphoreType.DMA,
        ],
    )(_make_segsum_body(base_row, nrows, rem))
    return f(h, ids, zeros)


def _out_body(p0_ref, p1_ref, p2_ref, p3_ref, wout_ref, bout_ref, out_ref):
    acc = (p0_ref[0:_B, :] + p0_ref[_B:2 * _B, :]
           + p1_ref[0:_B, :] + p1_ref[_B:2 * _B, :]
           + p2_ref[0:_B, :] + p2_ref[_B:2 * _B, :]
           + p3_ref[0:_B, :] + p3_ref[_B:2 * _B, :])
    out_ref[...] = (jnp.dot(acc, wout_ref[...],
                            preferred_element_type=jnp.float32)
                    + bout_ref[...])


def _out_layer(partials, Wout, bout):
    pspec = pl.BlockSpec((2 * _B, _H), lambda: (0, 0))
    return pl.pallas_call(
        _out_body,
        in_specs=[pspec, pspec, pspec, pspec,
                  pl.BlockSpec((_H, _O), lambda: (0, 0)),
                  pl.BlockSpec((1, _O), lambda: (0, 0))],
        out_specs=pl.BlockSpec((_B, _O), lambda: (0, 0)),
        out_shape=jax.ShapeDtypeStruct((_B, _O), jnp.float32),
    )(*partials, Wout, bout.reshape(1, _O))


def kernel(node_features, batch_vector, W1, b1, W2, b2, Wout, bout):
    ids = batch_vector.astype(jnp.int32)
    zeros = jnp.zeros((_B, _H), jnp.float32)
    partials = []
    base = 0
    for nrows in _SLICES:
        h_s = _mlp(node_features, W1, b1, W2, b2, base // _BN, nrows // _BN)
        partials.append(_segsum(h_s, ids, zeros, base, nrows))
        base += nrows
    return _out_layer(partials, Wout, bout)


# speedup vs baseline: 1.3250x; 1.3139x over previous
import functools

import jax
import jax.numpy as jnp
from jax import lax
from jax.experimental import pallas as pl
from jax.experimental.pallas import tpu as pltpu
from jax.experimental.pallas import tpu_sc as plsc

_N, _D, _H, _O, _B = 100000, 128, 128, 128, 1024
_BN = 2000
_SLICES = (26000, 26000, 26000, 22000)
_CH = 128
_NW = 32
_STRIPE = _B // 16
_NBUF = 3


def _mlp_body(x_ref, w1_ref, b1_ref, w2_ref, b2_ref, h_ref):
    x = x_ref[...]
    h = jnp.dot(x, w1_ref[...], preferred_element_type=jnp.float32)
    h = jnp.maximum(h + b1_ref[...], 0.0)
    h = jnp.dot(h, w2_ref[...], preferred_element_type=jnp.float32)
    h_ref[...] = jnp.maximum(h + b2_ref[...], 0.0)


def _mlp(x, W1, b1, W2, b2, base_blk, nb):
    return pl.pallas_call(
        _mlp_body,
        grid=(nb,),
        in_specs=[
            pl.BlockSpec((_BN, _D), lambda g: (g + base_blk, 0)),
            pl.BlockSpec((_D, _H), lambda g: (0, 0)),
            pl.BlockSpec((1, _H), lambda g: (0, 0)),
            pl.BlockSpec((_H, _H), lambda g: (0, 0)),
            pl.BlockSpec((1, _H), lambda g: (0, 0)),
        ],
        out_specs=pl.BlockSpec((_BN, _H), lambda g: (g, 0)),
        out_shape=jax.ShapeDtypeStruct((nb * _BN, _H), jnp.float32),
    )(x, W1, b1.reshape(1, _H), W2, b2.reshape(1, _H))


def _make_segsum_body(base_row, nrows, rem):
    nfull = nrows // _CH
    tmax = -(-nfull // _NW)
    ntail = nfull % _NW

    def body(h_hbm, ids_hbm, zeros_hbm, out_hbm,
             idx0, idx1, idx2, rows0, rows1, rows2, idx_r, rows_r, acc_sh,
             sl0, sl1, sl2, ss0, ss1, ss2):
        cid = lax.axis_index("c")
        sid = lax.axis_index("s")
        wid = sid * 2 + cid
        idx = (idx0, idx1, idx2)
        rows = (rows0, rows1, rows2)
        sem_l = (sl0, sl1, sl2)
        sem_s = (ss0, ss1, ss2)

        pltpu.sync_copy(zeros_hbm.at[pl.ds(sid * _STRIPE, _STRIPE)],
                        acc_sh.at[pl.ds(sid * _STRIPE, _STRIPE)])

        def valid(t):
            return t < tmax - 1 or ntail == 0

        def guard(t, fn):
            if valid(t):
                fn()
            else:
                pl.when(wid < ntail)(fn)

        def start_load(t, b):
            base = (wid + _NW * t) * _CH
            pltpu.async_copy(ids_hbm.at[pl.ds(base_row + base, _CH)],
                             idx[b], sem_l[b])
            pltpu.async_copy(h_hbm.at[pl.ds(base, _CH)], rows[b], sem_l[b])

        def issue_scatter(t, b):
            base = (wid + _NW * t) * _CH
            pltpu.make_async_copy(ids_hbm.at[pl.ds(base_row + base, _CH)],
                                  idx[b], sem_l[b]).wait()
            pltpu.make_async_copy(h_hbm.at[pl.ds(base, _CH)], rows[b],
                                  sem_l[b]).wait()
            pltpu.async_copy(rows[b], acc_sh.at[idx[b]], sem_s[b], add=True)

        def wait_scatter(t, b):
            pltpu.make_async_copy(rows[b], acc_sh.at[idx[b]],
                                  sem_s[b]).wait()

        guard(0, lambda: start_load(0, 0))
        plsc.subcore_barrier()

        for t in range(tmax):
            b = t % _NBUF
            if t >= 2:
                guard(t - 2, functools.partial(wait_scatter, t - 2,
                                               (t - 2) % _NBUF))
            if t + 1 < tmax:
                guard(t + 1, functools.partial(start_load, t + 1,
                                               (t + 1) % _NBUF))
            guard(t, functools.partial(issue_scatter, t, b))
        for t in (tmax - 2, tmax - 1):
            if t >= 0:
                guard(t, functools.partial(wait_scatter, t, t % _NBUF))

        if rem > 0:
            @pl.when(wid == _NW - 1)
            def _rem():
                base = nfull * _CH
                pltpu.sync_copy(ids_hbm.at[pl.ds(base_row + base, rem)],
                                idx_r)
                pltpu.sync_copy(h_hbm.at[pl.ds(base, rem)], rows_r)
                pltpu.sync_copy(rows_r, acc_sh.at[idx_r], add=True)

        plsc.subcore_barrier()
        out_base = cid * _B + sid * _STRIPE
        pltpu.sync_copy(acc_sh.at[pl.ds(sid * _STRIPE, _STRIPE)],
                        out_hbm.at[pl.ds(out_base, _STRIPE)])

    return body


def _segsum(h, ids, zeros, base_row, nrows):
    rem = nrows % _CH
    mesh = plsc.VectorSubcoreMesh(core_axis_name="c", subcore_axis_name="s")
    f = functools.partial(
        pl.kernel,
        mesh=mesh,
        out_type=jax.ShapeDtypeStruct((2 * _B, _H), jnp.float32),
        scratch_types=[
            pltpu.VMEM((_CH,), jnp.int32),
            pltpu.VMEM((_CH,), jnp.int32),
            pltpu.VMEM((_CH,), jnp.int32),
            pltpu.VMEM((_CH, _H), jnp.float32),
            pltpu.VMEM((_CH, _H), jnp.float32),
            pltpu.VMEM((_CH, _H), jnp.float32),
            pltpu.VMEM((max(rem, 8),), jnp.int32),
            pltpu.VMEM((max(rem, 8), _H), jnp.float32),
            pltpu.VMEM_SHARED((_B, _H), jnp.float32),
            pltpu.SemaphoreType.DMA,
            pltpu.SemaphoreType.DMA,
            pltpu.SemaphoreType.DMA,
            pltpu.SemaphoreType.DMA,
            pltpu.SemaphoreType.DMA,
            pltpu.SemaphoreType.DMA,
        ],
    )(_make_segsum_body(base_row, nrows, rem))
    return f(h, ids, zeros)


def _out_body(p0_ref, p1_ref, p2_ref, p3_ref, wout_ref, bout_ref, out_ref):
    acc = (p0_ref[0:_B, :] + p0_ref[_B:2 * _B, :]
           + p1_ref[0:_B, :] + p1_ref[_B:2 * _B, :]
           + p2_ref[0:_B, :] + p2_ref[_B:2 * _B, :]
           + p3_ref[0:_B, :] + p3_ref[_B:2 * _B, :])
    out_ref[...] = (jnp.dot(acc, wout_ref[...],
                            preferred_element_type=jnp.float32)
                    + bout_ref[...])


def _out_layer(partials, Wout, bout):
    pspec = pl.BlockSpec((2 * _B, _H), lambda: (0, 0))
    return pl.pallas_call(
        _out_body,
        in_specs=[pspec, pspec, pspec, pspec,
                  pl.BlockSpec((_H, _O), lambda: (0, 0)),
                  pl.BlockSpec((1, _O), lambda: (0, 0))],
        out_specs=pl.BlockSpec((_B, _O), lambda: (0, 0)),
        out_shape=jax.ShapeDtypeStruct((_B, _O), jnp.float32),
    )(*partials, Wout, bout.reshape(1, _O))


def kernel(node_features, batch_vector, W1, b1, W2, b2, Wout, bout):
    ids = batch_vector.astype(jnp.int32)
    zeros = jnp.zeros((_B, _H), jnp.float32)
    partials = []
    base = 0
    for nrows in _SLICES:
        h_s = _mlp(node_features, W1, b1, W2, b2, base // _BN, nrows // _BN)
        partials.append(_segsum(h_s, ids, zeros, base, nrows))
        base += nrows
    return _out_layer(partials, Wout, bout)
